# Initial kernel scaffold; baseline (speedup 1.0000x reference)
import jax
import jax.numpy as jnp
from jax.experimental import pallas as pl


def kernel(tgt_index, knn_dists, knn_key_feature, network_probs, network_select_probs, W_df, b_df, W_fc1a, b_fc1a, W_fc1b, b_fc1b, W_fc2a, b_fc2a, W_fc2b, b_fc2b):
    raise NotImplementedError("write your pallas kernel here")


# speedup vs baseline: 1.7661x; 1.7661x over previous
import functools

import jax
import jax.numpy as jnp
from jax import lax
from jax.experimental import pallas as pl
from jax.experimental.pallas import tpu as pltpu
from jax.experimental.pallas import tpu_sc as plsc

_B, _S, _V = 32, 8, 100000
_K = 32
_R = _B * _S
_NC, _NS, _L = 2, 16, 16
_NW = _NC * _NS
_RPW = _R // _NW
_VPAD = _V + _K


def _tc_body(tgt_ref, d_ref, kkf_ref, nsp_ref, w2aT_ref, b2a_ref, w2b1_ref,
             w1a_ref, b1a_ref, w1b_ref, b1b_ref, b2b1_ref,
             idx_out_ref, val_out_ref):
    tgt = tgt_ref[...]
    d = d_ref[...]
    lk = jnp.log(kkf_ref[...])
    ln = jnp.log(nsp_ref[...])

    noise = jnp.full_like(d, b1b_ref[0, 0])
    for c in range(4):
        h1c = jnp.tanh(lk * w1a_ref[c, 0] + ln * w1a_ref[c, 1] + b1a_ref[0, c])
        noise = noise + h1c * w1b_ref[0, c]

    jpos = lax.broadcasted_iota(jnp.int32, (_R, _K), 1)
    dup = jnp.zeros((_R, _K), jnp.bool_)
    for m in range(_K):
        eqm = tgt == tgt[:, m:m + 1]
        dup = dup | (eqm & (jpos > m))
    occ = ~dup
    occ_nz = occ & (tgt != 0)

    kj = lax.broadcasted_iota(jnp.int32, (_K, _K), 0)
    ki = lax.broadcasted_iota(jnp.int32, (_K, _K), 1)
    tri = (kj <= ki).astype(jnp.float32)
    lc = jax.lax.dot(occ_nz.astype(jnp.float32), tri,
                     precision=jax.lax.Precision.HIGHEST)

    feat = jnp.concatenate([d, lc], axis=-1)
    h2 = jnp.tanh(jax.lax.dot(feat, w2aT_ref[...],
                              precision=jax.lax.Precision.HIGHEST)
                  + b2a_ref[...])
    lam1 = jnp.sum(h2 * w2b1_ref[...], axis=-1, keepdims=True) + b2b1_ref[0, 0]
    tempe = jax.nn.sigmoid(lam1)

    logits = -d * tempe + noise
    mx = jnp.max(logits, axis=-1, keepdims=True)
    e = jnp.exp(logits - mx)
    probs = e / jnp.sum(e, axis=-1, keepdims=True)

    comb = jnp.zeros_like(d)
    for m in range(_K):
        eqm = (tgt == tgt[:, m:m + 1]).astype(jnp.float32)
        comb = comb + probs[:, m:m + 1] * eqm
    idx_out_ref[...] = jnp.where(occ, tgt, _V + jpos)
    val_out_ref[...] = jnp.where(occ, comb, 0.0)


_tc_compute = pl.pallas_call(
    _tc_body,
    out_shape=(
        jax.ShapeDtypeStruct((_R, _K), jnp.int32),
        jax.ShapeDtypeStruct((_R, _K), jnp.float32),
    ),
    in_specs=[
        pl.BlockSpec(memory_space=pltpu.VMEM),
        pl.BlockSpec(memory_space=pltpu.VMEM),
        pl.BlockSpec(memory_space=pltpu.VMEM),
        pl.BlockSpec(memory_space=pltpu.VMEM),
        pl.BlockSpec(memory_space=pltpu.VMEM),
        pl.BlockSpec(memory_space=pltpu.VMEM),
        pl.BlockSpec(memory_space=pltpu.VMEM),
        pl.BlockSpec(memory_space=pltpu.SMEM),
        pl.BlockSpec(memory_space=pltpu.SMEM),
        pl.BlockSpec(memory_space=pltpu.SMEM),
        pl.BlockSpec(memory_space=pltpu.SMEM),
        pl.BlockSpec(memory_space=pltpu.SMEM),
    ],
    out_specs=(
        pl.BlockSpec(memory_space=pltpu.VMEM),
        pl.BlockSpec(memory_space=pltpu.VMEM),
    ),
)


def _sc_scatter_body(idx_hbm, val_hbm, zeros_hbm, out_hbm, idx_v, val_v, acc_v):
    wid = lax.axis_index("s") * _NC + lax.axis_index("c")
    base = pl.multiple_of(wid * (_RPW * _K), _RPW * _K)
    pltpu.sync_copy(idx_hbm.at[pl.ds(base, _RPW * _K)], idx_v)
    pltpu.sync_copy(val_hbm.at[pl.ds(base, _RPW * _K)], val_v)
    pltpu.sync_copy(zeros_hbm, acc_v)
    zero16 = jnp.zeros((_L,), jnp.float32)
    for r in range(_RPW):
        for c in range(_K // _L):
            o = r * _K + c * _L
            plsc.addupdate_scatter(acc_v, [idx_v[pl.ds(o, _L)]],
                                   val_v[pl.ds(o, _L)])
        row = wid * _RPW + r
        off = pl.multiple_of(row * _V, _V)
        pltpu.sync_copy(acc_v.at[pl.ds(0, _V)], out_hbm.at[pl.ds(off, _V)])
        for c in range(_K // _L):
            o = r * _K + c * _L
            plsc.store_scatter(acc_v, [idx_v[pl.ds(o, _L)]], zero16)


@functools.cache
def _get_sc_scatter():
    return pl.kernel(
        _sc_scatter_body,
        out_type=jax.ShapeDtypeStruct((_R * _V,), jnp.float32),
        mesh=plsc.VectorSubcoreMesh(core_axis_name="c", subcore_axis_name="s",
                                    num_cores=_NC, num_subcores=_NS),
        scratch_types=[
            pltpu.VMEM((_RPW * _K,), jnp.int32),
            pltpu.VMEM((_RPW * _K,), jnp.float32),
            pltpu.VMEM((_VPAD,), jnp.float32),
        ],
        compiler_params=pltpu.CompilerParams(needs_layout_passes=False),
    )


def kernel(tgt_index, knn_dists, knn_key_feature, network_probs,
           network_select_probs, W_df, b_df, W_fc1a, b_fc1a, W_fc1b, b_fc1b,
           W_fc2a, b_fc2a, W_fc2b, b_fc2b):
    tgt = tgt_index.reshape(_R, _K).astype(jnp.int32)
    d = knn_dists.reshape(_R, _K)
    kkf = knn_key_feature.reshape(_R, _K)
    nsp = network_select_probs.reshape(_R, _K)
    idx, val = _tc_compute(
        tgt, d, kkf, nsp,
        W_fc2a.T, b_fc2a.reshape(1, -1), W_fc2b[1:2, :],
        W_fc1a, b_fc1a.reshape(1, -1), W_fc1b, b_fc1b.reshape(1, 1),
        b_fc2b[1:2].reshape(1, 1),
    )
    zeros = jnp.zeros((_VPAD,), jnp.float32)
    out = _get_sc_scatter()(idx.reshape(-1), val.reshape(-1), zeros)
    return out.reshape(_B, _S, _V)


# speedup vs baseline: 1.7840x; 1.0101x over previous
import functools

import jax
import jax.numpy as jnp
from jax import lax
from jax.experimental import pallas as pl
from jax.experimental.pallas import tpu as pltpu
from jax.experimental.pallas import tpu_sc as plsc

_B, _S, _V = 32, 8, 100000
_K = 32
_R = _B * _S
_NC, _NS, _L = 2, 16, 16
_NW = _NC * _NS
_RPW = _R // _NW
_VPAD = _V + _K


def _tc_body(tgt_ref, d_ref, kkf_ref, nsp_ref, w2a_ref, b2a_ref, w2b1_ref,
             w1a_ref, b1a_ref, w1b_ref, b1b_ref, b2b1_ref,
             idx_out_ref, val_out_ref):
    tgt = tgt_ref[...]
    d = d_ref[...]
    lk = jnp.log(kkf_ref[...])
    ln = jnp.log(nsp_ref[...])

    noise = jnp.full_like(d, b1b_ref[0, 0])
    for c in range(4):
        h1c = jnp.tanh(lk * w1a_ref[c, 0] + ln * w1a_ref[c, 1] + b1a_ref[0, c])
        noise = noise + h1c * w1b_ref[0, c]

    jpos = lax.broadcasted_iota(jnp.int32, (_R, _K), 1)
    dup = jnp.zeros((_R, _K), jnp.bool_)
    for m in range(_K):
        eqm = tgt == tgt[:, m:m + 1]
        dup = dup | (eqm & (jpos > m))
    occ = ~dup
    occ_nz = occ & (tgt != 0)

    kj = lax.broadcasted_iota(jnp.int32, (_K, _K), 0)
    ki = lax.broadcasted_iota(jnp.int32, (_K, _K), 1)
    tri = (kj <= ki).astype(jnp.float32)
    lc = jax.lax.dot(occ_nz.astype(jnp.float32), tri,
                     precision=jax.lax.Precision.HIGHEST)

    feat = jnp.concatenate([d, lc], axis=-1)
    h2 = jnp.tanh(lax.dot_general(feat, w2a_ref[...], (((1,), (1,)), ((), ())),
                                  precision=jax.lax.Precision.HIGHEST)
                  + b2a_ref[...])
    lam1 = jnp.sum(h2 * w2b1_ref[...], axis=-1, keepdims=True) + b2b1_ref[0, 0]
    tempe = jax.nn.sigmoid(lam1)

    logits = -d * tempe + noise
    mx = jnp.max(logits, axis=-1, keepdims=True)
    e = jnp.exp(logits - mx)
    probs = e / jnp.sum(e, axis=-1, keepdims=True)

    comb = jnp.zeros_like(d)
    for m in range(_K):
        eqm = (tgt == tgt[:, m:m + 1]).astype(jnp.float32)
        comb = comb + probs[:, m:m + 1] * eqm
    idx_out_ref[...] = jnp.where(occ, tgt, _V + jpos)
    val_out_ref[...] = jnp.where(occ, comb, 0.0)


_tc_compute = pl.pallas_call(
    _tc_body,
    out_shape=(
        jax.ShapeDtypeStruct((_R, _K), jnp.int32),
        jax.ShapeDtypeStruct((_R, _K), jnp.float32),
    ),
    in_specs=[
        pl.BlockSpec(memory_space=pltpu.VMEM),
        pl.BlockSpec(memory_space=pltpu.VMEM),
        pl.BlockSpec(memory_space=pltpu.VMEM),
        pl.BlockSpec(memory_space=pltpu.VMEM),
        pl.BlockSpec(memory_space=pltpu.VMEM),
        pl.BlockSpec(memory_space=pltpu.VMEM),
        pl.BlockSpec(memory_space=pltpu.VMEM),
        pl.BlockSpec(memory_space=pltpu.SMEM),
        pl.BlockSpec(memory_space=pltpu.SMEM),
        pl.BlockSpec(memory_space=pltpu.SMEM),
        pl.BlockSpec(memory_space=pltpu.SMEM),
        pl.BlockSpec(memory_space=pltpu.SMEM),
    ],
    out_specs=(
        pl.BlockSpec(memory_space=pltpu.VMEM),
        pl.BlockSpec(memory_space=pltpu.VMEM),
    ),
)


def _sc_scatter_body(idx_hbm, val_hbm, zeros_hbm, out_hbm, idx_v, val_v, acc_v):
    wid = lax.axis_index("s") * _NC + lax.axis_index("c")
    base = pl.multiple_of(wid * (_RPW * _K), _RPW * _K)
    pltpu.sync_copy(idx_hbm.at[pl.ds(base, _RPW * _K)], idx_v)
    pltpu.sync_copy(val_hbm.at[pl.ds(base, _RPW * _K)], val_v)
    pltpu.sync_copy(zeros_hbm, acc_v)
    zero16 = jnp.zeros((_L,), jnp.float32)
    for r in range(_RPW):
        for c in range(_K // _L):
            o = r * _K + c * _L
            plsc.addupdate_scatter(acc_v, [idx_v[pl.ds(o, _L)]],
                                   val_v[pl.ds(o, _L)])
        row = wid * _RPW + r
        pltpu.sync_copy(acc_v.at[pl.ds(0, _V)], out_hbm.at[row // _S, row % _S])
        for c in range(_K // _L):
            o = r * _K + c * _L
            plsc.store_scatter(acc_v, [idx_v[pl.ds(o, _L)]], zero16)


@functools.cache
def _get_sc_scatter():
    return pl.kernel(
        _sc_scatter_body,
        out_type=jax.ShapeDtypeStruct((_B, _S, _V), jnp.float32),
        mesh=plsc.VectorSubcoreMesh(core_axis_name="c", subcore_axis_name="s",
                                    num_cores=_NC, num_subcores=_NS),
        scratch_types=[
            pltpu.VMEM((_RPW * _K,), jnp.int32),
            pltpu.VMEM((_RPW * _K,), jnp.float32),
            pltpu.VMEM((_VPAD,), jnp.float32),
        ],
        compiler_params=pltpu.CompilerParams(needs_layout_passes=False,
                                             use_tc_tiling_on_sc=False),
    )


def kernel(tgt_index, knn_dists, knn_key_feature, network_probs,
           network_select_probs, W_df, b_df, W_fc1a, b_fc1a, W_fc1b, b_fc1b,
           W_fc2a, b_fc2a, W_fc2b, b_fc2b):
    tgt = tgt_index.reshape(_R, _K).astype(jnp.int32)
    d = knn_dists.reshape(_R, _K)
    kkf = knn_key_feature.reshape(_R, _K)
    nsp = network_select_probs.reshape(_R, _K)
    idx, val = _tc_compute(
        tgt, d, kkf, nsp,
        W_fc2a, b_fc2a.reshape(1, -1), W_fc2b[1:2, :],
        W_fc1a, b_fc1a.reshape(1, -1), W_fc1b, b_fc1b.reshape(1, 1),
        b_fc2b[1:2].reshape(1, 1),
    )
    zeros = jnp.zeros((_VPAD,), jnp.float32)
    return _get_sc_scatter()(idx.reshape(-1), val.reshape(-1), zeros)


# speedup vs baseline: 5.0163x; 2.8119x over previous
import functools

import jax
import jax.numpy as jnp
from jax import lax
from jax.experimental import pallas as pl
from jax.experimental.pallas import tpu as pltpu
from jax.experimental.pallas import tpu_sc as plsc

_B, _S, _V = 32, 8, 100000
_K = 32
_R = _B * _S
_NC, _NS, _L = 2, 16, 16
_NW = _NC * _NS
_RPW = _R // _NW
_PADW = 128


def _tc_body(tgt_ref, d_ref, kkf_ref, nsp_ref, w2a_ref, b2a_ref, w2b_ref,
             w1a_ref, b1a_ref, w1b_ref, b1b_ref, b2b_ref,
             idx_out_ref, val_out_ref):
    tgt = tgt_ref[...]
    d = d_ref[...]
    lk = jnp.log(kkf_ref[...])
    ln = jnp.log(nsp_ref[...])

    noise = jnp.full_like(d, b1b_ref[0, 0])
    for c in range(4):
        h1c = jnp.tanh(lk * w1a_ref[c, 0] + ln * w1a_ref[c, 1] + b1a_ref[0, c])
        noise = noise + h1c * w1b_ref[0, c]

    jpos = lax.broadcasted_iota(jnp.int32, (_R, _K), 1)
    dup = jnp.zeros((_R, _K), jnp.bool_)
    for m in range(_K):
        eqm = tgt == tgt[:, m:m + 1]
        dup = dup | (eqm & (jpos > m))
    occ = ~dup
    occ_nz = occ & (tgt != 0)

    kj = lax.broadcasted_iota(jnp.int32, (_K, _K), 0)
    ki = lax.broadcasted_iota(jnp.int32, (_K, _K), 1)
    tri = (kj <= ki).astype(jnp.float32)
    lc = jax.lax.dot(occ_nz.astype(jnp.float32), tri,
                     precision=jax.lax.Precision.HIGHEST)

    feat = jnp.concatenate([d, lc], axis=-1)
    h2 = jnp.tanh(lax.dot_general(feat, w2a_ref[...], (((1,), (1,)), ((), ())),
                                  precision=jax.lax.Precision.HIGHEST)
                  + b2a_ref[...])
    lam1 = jnp.sum(h2 * w2b_ref[1:2, :], axis=-1, keepdims=True) + b2b_ref[1]
    tempe = jax.nn.sigmoid(lam1)

    logits = -d * tempe + noise
    mx = jnp.max(logits, axis=-1, keepdims=True)
    e = jnp.exp(logits - mx)
    probs = e / jnp.sum(e, axis=-1, keepdims=True)

    comb = jnp.zeros_like(d)
    for m in range(_K):
        eqm = (tgt == tgt[:, m:m + 1]).astype(jnp.float32)
        comb = comb + probs[:, m:m + 1] * eqm
    idx = jnp.where(occ, tgt, -1)
    val = jnp.where(occ, comb, 0.0)
    idx_out_ref[...] = jnp.concatenate(
        [idx, jnp.full((_R, _PADW - _K), -1, jnp.int32)], axis=-1)
    val_out_ref[...] = jnp.concatenate(
        [val, jnp.zeros((_R, _PADW - _K), jnp.float32)], axis=-1)


_tc_compute = pl.pallas_call(
    _tc_body,
    out_shape=(
        jax.ShapeDtypeStruct((_R, _PADW), jnp.int32),
        jax.ShapeDtypeStruct((_R, _PADW), jnp.float32),
    ),
    in_specs=[
        pl.BlockSpec(memory_space=pltpu.VMEM),
        pl.BlockSpec(memory_space=pltpu.VMEM),
        pl.BlockSpec(memory_space=pltpu.VMEM),
        pl.BlockSpec(memory_space=pltpu.VMEM),
        pl.BlockSpec(memory_space=pltpu.VMEM),
        pl.BlockSpec(memory_space=pltpu.VMEM),
        pl.BlockSpec(memory_space=pltpu.VMEM),
        pl.BlockSpec(memory_space=pltpu.SMEM),
        pl.BlockSpec(memory_space=pltpu.SMEM),
        pl.BlockSpec(memory_space=pltpu.SMEM),
        pl.BlockSpec(memory_space=pltpu.SMEM),
        pl.BlockSpec(memory_space=pltpu.SMEM),
    ],
    out_specs=(
        pl.BlockSpec(memory_space=pltpu.VMEM),
        pl.BlockSpec(memory_space=pltpu.VMEM),
    ),
)


_CW = 6144
_NCH = 16
_LW = _V - _NCH * _CW


def _sc_scatter_body(idx_hbm, val_hbm, zeros_hbm, out_hbm,
                     idx_v, val_v, acc_a, acc_b, acc_l, sem_a, sem_b, sem_l):
    wid = lax.axis_index("s") * _NC + lax.axis_index("c")
    r0 = pl.multiple_of(wid * _RPW, _RPW)
    h_a = pltpu.async_copy(zeros_hbm.at[:, pl.ds(0, _CW)], acc_a, sem_a)
    h_b = pltpu.async_copy(zeros_hbm.at[:, pl.ds(0, _CW)], acc_b, sem_b)
    h_l = pltpu.async_copy(zeros_hbm.at[:, pl.ds(_CW, _LW)], acc_l, sem_l)
    pltpu.sync_copy(idx_hbm.at[pl.ds(r0, _RPW), :], idx_v)
    pltpu.sync_copy(val_hbm.at[pl.ds(r0, _RPW), :], val_v)
    zero16 = jnp.zeros((_L,), jnp.float32)

    def scatter_pass(acc, lo, width, zero_pass):
        for r in range(_S):
            s_vec = jnp.full((_L,), r, jnp.int32)
            for c in range(_K // _L):
                v16 = idx_v[r, pl.ds(c * _L, _L)]
                m = (v16 >= lo) & (v16 < lo + width)
                if zero_pass:
                    plsc.store_scatter(acc, [s_vec, v16 - lo], zero16, mask=m)
                else:
                    plsc.addupdate_scatter(acc, [s_vec, v16 - lo],
                                           val_v[r, pl.ds(c * _L, _L)], mask=m)

    accs, sems, pend = (acc_a, acc_b), (sem_a, sem_b), [h_a, h_b]
    pend_lo = [None, None]
    for ci in range(_NCH):
        pp = ci % 2
        acc = accs[pp]
        pend[pp].wait()
        if pend_lo[pp] is not None:
            scatter_pass(acc, pend_lo[pp], _CW, True)
        lo = ci * _CW
        scatter_pass(acc, lo, _CW, False)
        pend[pp] = pltpu.async_copy(acc, out_hbm.at[wid, :, pl.ds(lo, _CW)],
                                    sems[pp])
        pend_lo[pp] = lo
    h_l.wait()
    lo = _NCH * _CW
    scatter_pass(acc_l, lo, _LW, False)
    h_l = pltpu.async_copy(acc_l, out_hbm.at[wid, :, pl.ds(lo, _LW)], sem_l)
    pend[0].wait()
    pend[1].wait()
    h_l.wait()


@functools.cache
def _get_sc_scatter():
    return pl.kernel(
        _sc_scatter_body,
        out_type=jax.ShapeDtypeStruct((_B, _S, _V), jnp.float32),
        mesh=plsc.VectorSubcoreMesh(core_axis_name="c", subcore_axis_name="s",
                                    num_cores=_NC, num_subcores=_NS),
        scratch_types=[
            pltpu.VMEM((_RPW, _PADW), jnp.int32),
            pltpu.VMEM((_RPW, _PADW), jnp.float32),
            pltpu.VMEM((_S, _CW), jnp.float32),
            pltpu.VMEM((_S, _CW), jnp.float32),
            pltpu.VMEM((_S, _LW), jnp.float32),
            pltpu.SemaphoreType.DMA,
            pltpu.SemaphoreType.DMA,
            pltpu.SemaphoreType.DMA,
        ],
        compiler_params=pltpu.CompilerParams(needs_layout_passes=False),
    )


def kernel(tgt_index, knn_dists, knn_key_feature, network_probs,
           network_select_probs, W_df, b_df, W_fc1a, b_fc1a, W_fc1b, b_fc1b,
           W_fc2a, b_fc2a, W_fc2b, b_fc2b):
    tgt = tgt_index.reshape(_R, _K).astype(jnp.int32)
    d = knn_dists.reshape(_R, _K)
    kkf = knn_key_feature.reshape(_R, _K)
    nsp = network_select_probs.reshape(_R, _K)
    idx, val = _tc_compute(
        tgt, d, kkf, nsp,
        W_fc2a, b_fc2a.reshape(1, -1), W_fc2b,
        W_fc1a, b_fc1a.reshape(1, -1), W_fc1b, b_fc1b.reshape(1, 1),
        b_fc2b,
    )
    zeros = jnp.zeros((_S, _CW + _LW), jnp.float32)
    return _get_sc_scatter()(idx, val, zeros)
